# initial kernel scaffold (unmeasured)
import jax
import jax.numpy as jnp
from jax import lax
from jax.experimental import pallas as pl
from jax.experimental.pallas import tpu as pltpu

N_DEV = 8
SQ = 2048
D_MODEL = 1024
H_LOC = 8
DH = 128
QB = 512
KW = 1024
N_QB = SQ // QB
CHUNK = SQ // N_DEV
SCALE = 0.08838834764831843
WINDOW = 128


def kernel(x, Wq, K_ext, V_ext, Wo):
    x2 = x.reshape(SQ, D_MODEL)
    K2 = K_ext.reshape(SQ, H_LOC * DH)
    V2 = V_ext.reshape(SQ, H_LOC * DH)

    def body(x_ref, wq_ref, k_ref, v_ref, wo_ref, out_ref,
             wq_v, wo_v, q_sc, ctx_sc, recv_buf,
             dma_sems, send_sems, recv_sems):
        i = lax.axis_index("i")
        left = lax.rem(i + N_DEV - 1, N_DEV)
        right = lax.rem(i + 1, N_DEV)

        wq_dma = pltpu.make_async_copy(
            wq_ref.at[:, pl.ds(i * D_MODEL, D_MODEL)], wq_v, dma_sems.at[0])
        wq_dma.start()
        wo_dma = pltpu.make_async_copy(
            wo_ref.at[pl.ds(i * D_MODEL, D_MODEL), :], wo_v, dma_sems.at[1])
        wo_dma.start()

        barrier_sem = pltpu.get_barrier_semaphore()
        pl.semaphore_signal(barrier_sem, inc=1, device_id=(left,),
                            device_id_type=pl.DeviceIdType.MESH)
        pl.semaphore_signal(barrier_sem, inc=1, device_id=(right,),
                            device_id_type=pl.DeviceIdType.MESH)
        pl.semaphore_wait(barrier_sem, 2)

        wq_dma.wait()
        q_sc[:, :] = jnp.dot(x_ref[:, :], wq_v[:, :],
                             preferred_element_type=jnp.float32)

        for qb in range(N_QB):
            r0 = qb * QB
            kw = min(max(r0 - 256, 0), SQ - KW)
            qi = r0 + lax.broadcasted_iota(jnp.int32, (QB, KW), 0)
            ki = kw + lax.broadcasted_iota(jnp.int32, (QB, KW), 1)
            mask = jnp.abs(qi - ki) <= WINDOW
            for h in range(H_LOC):
                c0 = h * DH
                q_h = q_sc[r0:r0 + QB, c0:c0 + DH]
                k_h = k_ref[kw:kw + KW, c0:c0 + DH]
                v_h = v_ref[kw:kw + KW, c0:c0 + DH]
                s = lax.dot_general(
                    q_h, k_h, (((1,), (1,)), ((), ())),
                    preferred_element_type=jnp.float32) * SCALE
                s = jnp.where(mask, s, -1e9)
                m = jnp.max(s, axis=1, keepdims=True)
                w = jnp.exp(s - m)
                w = w / jnp.sum(w, axis=1, keepdims=True)
                ctx_sc[r0:r0 + QB, c0:c0 + DH] = jnp.dot(
                    w, v_h, preferred_element_type=jnp.float32)

        wo_dma.wait()
        out_ref[:, :] = jnp.dot(ctx_sc[:, :], wo_v[:, :],
                                preferred_element_type=jnp.float32)

        for s in range(N_DEV - 1):
            send_c = lax.rem(i - s + 2 * N_DEV, N_DEV)
            recv_c = lax.rem(i - s - 1 + 2 * N_DEV, N_DEV)
            rdma = pltpu.make_async_remote_copy(
                src_ref=out_ref.at[pl.ds(send_c * CHUNK, CHUNK), :],
                dst_ref=recv_buf.at[s],
                send_sem=send_sems.at[s],
                recv_sem=recv_sems.at[s],
                device_id=(right,),
                device_id_type=pl.DeviceIdType.MESH,
            )
            rdma.start()
            rdma.wait()
            rows = pl.ds(recv_c * CHUNK, CHUNK)
            out_ref[rows, :] = out_ref[rows, :] + recv_buf[s, :, :]

        for s in range(N_DEV - 1):
            send_c = lax.rem(i + 1 - s + 2 * N_DEV, N_DEV)
            rows = pl.ds(send_c * CHUNK, CHUNK)
            rdma = pltpu.make_async_remote_copy(
                src_ref=out_ref.at[rows, :],
                dst_ref=out_ref.at[rows, :],
                send_sem=send_sems.at[N_DEV - 1 + s],
                recv_sem=recv_sems.at[N_DEV - 1 + s],
                device_id=(right,),
                device_id_type=pl.DeviceIdType.MESH,
            )
            rdma.start()
            rdma.wait()

    out = pl.pallas_call(
        body,
        out_shape=jax.ShapeDtypeStruct((SQ, D_MODEL), jnp.float32),
        in_specs=[
            pl.BlockSpec(memory_space=pltpu.VMEM),
            pl.BlockSpec(memory_space=pltpu.ANY),
            pl.BlockSpec(memory_space=pltpu.VMEM),
            pl.BlockSpec(memory_space=pltpu.VMEM),
            pl.BlockSpec(memory_space=pltpu.ANY),
        ],
        out_specs=pl.BlockSpec(memory_space=pltpu.VMEM),
        scratch_shapes=[
            pltpu.VMEM((D_MODEL, D_MODEL), jnp.float32),
            pltpu.VMEM((D_MODEL, D_MODEL), jnp.float32),
            pltpu.VMEM((SQ, D_MODEL), jnp.float32),
            pltpu.VMEM((SQ, D_MODEL), jnp.float32),
            pltpu.VMEM((N_DEV - 1, CHUNK, D_MODEL), jnp.float32),
            pltpu.SemaphoreType.DMA((2,)),
            pltpu.SemaphoreType.DMA((2 * (N_DEV - 1),)),
            pltpu.SemaphoreType.DMA((2 * (N_DEV - 1),)),
        ],
        compiler_params=pltpu.CompilerParams(
            collective_id=0,
            vmem_limit_bytes=120 * 1024 * 1024,
        ),
    )(x2, Wq, K2, V2, Wo)
    return out.reshape(1, SQ, D_MODEL)


# baseline (device time: 257522 ns/iter reference)
import jax
import jax.numpy as jnp
from jax import lax
from jax.experimental import pallas as pl
from jax.experimental.pallas import tpu as pltpu

N_DEV = 8
SQ = 2048
D_MODEL = 1024
H_LOC = 8
DH = 128
QB = 512
KW = 1024
N_QB = SQ // QB
CHUNK = SQ // N_DEV
SCALE = 0.08838834764831843
WINDOW = 128


def kernel(x, Wq, K_ext, V_ext, Wo):
    x2 = x.reshape(SQ, D_MODEL)
    K2 = K_ext.reshape(SQ, H_LOC * DH)
    V2 = V_ext.reshape(SQ, H_LOC * DH)

    def body(x_ref, wq_ref, k_ref, v_ref, wo_ref, out_ref,
             wq_v, wo_v, q_sc, ctx_sc, recv_buf,
             dma_sems, send_sems, recv_sems):
        i = lax.axis_index("i")
        left = lax.rem(i + N_DEV - 1, N_DEV)
        right = lax.rem(i + 1, N_DEV)

        wq_dma = pltpu.make_async_copy(
            wq_ref.at[:, pl.ds(i * D_MODEL, D_MODEL)], wq_v, dma_sems.at[0])
        wq_dma.start()
        wo_dma = pltpu.make_async_copy(
            wo_ref.at[pl.ds(i * D_MODEL, D_MODEL), :], wo_v, dma_sems.at[1])
        wo_dma.start()

        barrier_sem = pltpu.get_barrier_semaphore()
        pl.semaphore_signal(barrier_sem, inc=1, device_id=(left,),
                            device_id_type=pl.DeviceIdType.MESH)
        pl.semaphore_signal(barrier_sem, inc=1, device_id=(right,),
                            device_id_type=pl.DeviceIdType.MESH)
        pl.semaphore_wait(barrier_sem, 2)

        wq_dma.wait()
        q_sc[:, :] = jnp.dot(x_ref[:, :], wq_v[:, :],
                             preferred_element_type=jnp.float32)

        for qb in range(N_QB):
            r0 = qb * QB
            kw = min(max(r0 - 256, 0), SQ - KW)
            qi = r0 + lax.broadcasted_iota(jnp.int32, (QB, KW), 0)
            ki = kw + lax.broadcasted_iota(jnp.int32, (QB, KW), 1)
            mask = jnp.abs(qi - ki) <= WINDOW
            for h in range(H_LOC):
                c0 = h * DH
                q_h = q_sc[r0:r0 + QB, c0:c0 + DH]
                k_h = k_ref[kw:kw + KW, c0:c0 + DH]
                v_h = v_ref[kw:kw + KW, c0:c0 + DH]
                s = lax.dot_general(
                    q_h, k_h, (((1,), (1,)), ((), ())),
                    preferred_element_type=jnp.float32) * SCALE
                s = jnp.where(mask, s, -1e9)
                m = jnp.max(s, axis=1, keepdims=True)
                w = jnp.exp(s - m)
                w = w / jnp.sum(w, axis=1, keepdims=True)
                ctx_sc[r0:r0 + QB, c0:c0 + DH] = jnp.dot(
                    w, v_h, preferred_element_type=jnp.float32)

        wo_dma.wait()
        out_ref[:, :] = jnp.dot(ctx_sc[:, :], wo_v[:, :],
                                preferred_element_type=jnp.float32)

        for s in range(N_DEV - 1):
            send_c = lax.rem(i - s + 2 * N_DEV, N_DEV)
            recv_c = lax.rem(i - s - 1 + 2 * N_DEV, N_DEV)
            rdma = pltpu.make_async_remote_copy(
                src_ref=out_ref.at[pl.ds(send_c * CHUNK, CHUNK), :],
                dst_ref=recv_buf.at[s],
                send_sem=send_sems.at[s],
                recv_sem=recv_sems.at[s],
                device_id=(right,),
                device_id_type=pl.DeviceIdType.MESH,
            )
            rdma.start()
            rdma.wait()
            rows = pl.ds(recv_c * CHUNK, CHUNK)
            out_ref[rows, :] = out_ref[rows, :] + recv_buf[s, :, :]

        for s in range(N_DEV - 1):
            send_c = lax.rem(i + 1 - s + 2 * N_DEV, N_DEV)
            rows = pl.ds(send_c * CHUNK, CHUNK)
            rdma = pltpu.make_async_remote_copy(
                src_ref=out_ref.at[rows, :],
                dst_ref=out_ref.at[rows, :],
                send_sem=send_sems.at[N_DEV - 1 + s],
                recv_sem=recv_sems.at[N_DEV - 1 + s],
                device_id=(right,),
                device_id_type=pl.DeviceIdType.MESH,
            )
            rdma.start()
            rdma.wait()

    out = pl.pallas_call(
        body,
        out_shape=jax.ShapeDtypeStruct((SQ, D_MODEL), jnp.float32),
        in_specs=[
            pl.BlockSpec(memory_space=pltpu.VMEM),
            pl.BlockSpec(memory_space=pl.ANY),
            pl.BlockSpec(memory_space=pltpu.VMEM),
            pl.BlockSpec(memory_space=pltpu.VMEM),
            pl.BlockSpec(memory_space=pl.ANY),
        ],
        out_specs=pl.BlockSpec(memory_space=pltpu.VMEM),
        scratch_shapes=[
            pltpu.VMEM((D_MODEL, D_MODEL), jnp.float32),
            pltpu.VMEM((D_MODEL, D_MODEL), jnp.float32),
            pltpu.VMEM((SQ, D_MODEL), jnp.float32),
            pltpu.VMEM((SQ, D_MODEL), jnp.float32),
            pltpu.VMEM((N_DEV - 1, CHUNK, D_MODEL), jnp.float32),
            pltpu.SemaphoreType.DMA((2,)),
            pltpu.SemaphoreType.DMA((2 * (N_DEV - 1),)),
            pltpu.SemaphoreType.DMA((2 * (N_DEV - 1),)),
        ],
        compiler_params=pltpu.CompilerParams(
            collective_id=0,
            vmem_limit_bytes=120 * 1024 * 1024,
        ),
    )(x2, Wq, K2, V2, Wo)
    return out.reshape(1, SQ, D_MODEL)


# device time: 162403 ns/iter; 1.5857x vs baseline; 1.5857x over previous
import jax
import jax.numpy as jnp
from jax import lax
from jax.experimental import pallas as pl
from jax.experimental.pallas import tpu as pltpu

N_DEV = 8
SQ = 2048
D_MODEL = 1024
H_LOC = 8
DH = 128
QB = 512
KW = 1024
N_QB = SQ // QB
CHUNK = SQ // N_DEV
SCALE = 0.08838834764831843
WINDOW = 128


def kernel(x, Wq, K_ext, V_ext, Wo):
    x2 = x.reshape(SQ, D_MODEL)
    K2 = K_ext.reshape(SQ, H_LOC * DH)
    V2 = V_ext.reshape(SQ, H_LOC * DH)

    def body(x_ref, wq_ref, k_ref, v_ref, wo_ref, out_ref,
             wq_v, wo_v, q_sc, ctx_sc, comm, rs_recv,
             dma_sems, send_sems, recv_sems):
        i = lax.axis_index("i")
        p = lax.rem(i, 4)
        bx = jnp.where((p == 1) | (p == 2), 1, 0)
        by = jnp.where(p >= 2, 1, 0)
        bz = i // 4
        part_x = bz * 4 + (p ^ 1)
        part_y = bz * 4 + (3 - p)
        part_z = i ^ 4

        wq_dma = pltpu.make_async_copy(
            wq_ref.at[:, pl.ds(i * D_MODEL, D_MODEL)], wq_v, dma_sems.at[0])
        wq_dma.start()
        wo_dma = pltpu.make_async_copy(
            wo_ref.at[pl.ds(i * D_MODEL, D_MODEL), :], wo_v, dma_sems.at[1])
        wo_dma.start()

        barrier_sem = pltpu.get_barrier_semaphore()
        for prt in (part_x, part_y, part_z):
            pl.semaphore_signal(barrier_sem, inc=1, device_id=(prt,),
                                device_id_type=pl.DeviceIdType.MESH)
        pl.semaphore_wait(barrier_sem, 3)

        wq_dma.wait()
        q_sc[:, :] = jnp.dot(x_ref[:, :], wq_v[:, :],
                             preferred_element_type=jnp.float32)

        for qb in range(N_QB):
            r0 = qb * QB
            kw = min(max(r0 - 256, 0), SQ - KW)
            qi = r0 + lax.broadcasted_iota(jnp.int32, (QB, KW), 0)
            ki = kw + lax.broadcasted_iota(jnp.int32, (QB, KW), 1)
            mask = jnp.abs(qi - ki) <= WINDOW
            for h in range(H_LOC):
                c0 = h * DH
                q_h = q_sc[r0:r0 + QB, c0:c0 + DH]
                k_h = k_ref[kw:kw + KW, c0:c0 + DH]
                v_h = v_ref[kw:kw + KW, c0:c0 + DH]
                s = lax.dot_general(
                    q_h, k_h, (((1,), (1,)), ((), ())),
                    preferred_element_type=jnp.float32) * SCALE
                s = jnp.where(mask, s, -1e9)
                m = jnp.max(s, axis=1, keepdims=True)
                w = jnp.exp(s - m)
                w = w / jnp.sum(w, axis=1, keepdims=True)
                ctx_sc[r0:r0 + QB, c0:c0 + DH] = jnp.dot(
                    w, v_h, preferred_element_type=jnp.float32)

        wo_dma.wait()
        comm[:, :] = jnp.dot(ctx_sc[:, :], wo_v[:, :],
                             preferred_element_type=jnp.float32
                             ).astype(jnp.bfloat16)

        half = SQ // 2
        quar = SQ // 4
        eigh = SQ // 8
        rs_plan = [
            (part_x, bx * half, (1 - bx) * half, half, 0),
            (part_y, bx * half + by * quar,
             bx * half + (1 - by) * quar, quar, half),
            (part_z, bx * half + by * quar + bz * eigh,
             bx * half + by * quar + (1 - bz) * eigh, eigh, half + quar),
        ]
        for ph, (prt, keep_off, send_off, size, slot) in enumerate(rs_plan):
            rdma = pltpu.make_async_remote_copy(
                src_ref=comm.at[pl.ds(send_off, size), :],
                dst_ref=rs_recv.at[pl.ds(slot, size), :],
                send_sem=send_sems.at[ph],
                recv_sem=recv_sems.at[ph],
                device_id=(prt,),
                device_id_type=pl.DeviceIdType.MESH,
            )
            rdma.start()
            rdma.wait()
            rows = pl.ds(keep_off, size)
            comm[rows, :] = comm[rows, :] + rs_recv[pl.ds(slot, size), :]

        ag_plan = [
            (part_z, bx * half + by * quar + bz * eigh, eigh),
            (part_y, bx * half + by * quar, quar),
            (part_x, bx * half, half),
        ]
        for ph, (prt, off, size) in enumerate(ag_plan):
            rows = pl.ds(off, size)
            rdma = pltpu.make_async_remote_copy(
                src_ref=comm.at[rows, :],
                dst_ref=comm.at[rows, :],
                send_sem=send_sems.at[3 + ph],
                recv_sem=recv_sems.at[3 + ph],
                device_id=(prt,),
                device_id_type=pl.DeviceIdType.MESH,
            )
            rdma.start()
            rdma.wait()

        out_ref[:, :] = comm[:, :].astype(jnp.float32)

    out = pl.pallas_call(
        body,
        out_shape=jax.ShapeDtypeStruct((SQ, D_MODEL), jnp.float32),
        in_specs=[
            pl.BlockSpec(memory_space=pltpu.VMEM),
            pl.BlockSpec(memory_space=pl.ANY),
            pl.BlockSpec(memory_space=pltpu.VMEM),
            pl.BlockSpec(memory_space=pltpu.VMEM),
            pl.BlockSpec(memory_space=pl.ANY),
        ],
        out_specs=pl.BlockSpec(memory_space=pltpu.VMEM),
        scratch_shapes=[
            pltpu.VMEM((D_MODEL, D_MODEL), jnp.float32),
            pltpu.VMEM((D_MODEL, D_MODEL), jnp.float32),
            pltpu.VMEM((SQ, D_MODEL), jnp.float32),
            pltpu.VMEM((SQ, D_MODEL), jnp.float32),
            pltpu.VMEM((SQ, D_MODEL), jnp.bfloat16),
            pltpu.VMEM((SQ // 2 + SQ // 4 + SQ // 8, D_MODEL),
                       jnp.bfloat16),
            pltpu.SemaphoreType.DMA((2,)),
            pltpu.SemaphoreType.DMA((6,)),
            pltpu.SemaphoreType.DMA((6,)),
        ],
        compiler_params=pltpu.CompilerParams(
            collective_id=0,
            vmem_limit_bytes=120 * 1024 * 1024,
        ),
    )(x2, Wq, K2, V2, Wo)
    return out.reshape(1, SQ, D_MODEL)


# device time: 161386 ns/iter; 1.5957x vs baseline; 1.0063x over previous
import jax
import jax.numpy as jnp
from jax import lax
from jax.experimental import pallas as pl
from jax.experimental.pallas import tpu as pltpu

N_DEV = 8
SQ = 2048
D_MODEL = 1024
H_LOC = 8
DH = 128
QB = 512
KW = 1024
N_QB = SQ // QB
CHUNK = SQ // N_DEV
SCALE = 0.08838834764831843
WINDOW = 128


def kernel(x, Wq, K_ext, V_ext, Wo):
    x2 = x.reshape(SQ, D_MODEL)
    K2 = K_ext.reshape(SQ, H_LOC * DH)
    V2 = V_ext.reshape(SQ, H_LOC * DH)

    def body(x_ref, wq_ref, k_ref, v_ref, wo_ref, out_ref,
             wq_v, wo_v, q_sc, ctx_sc, comm, rs_recv,
             dma_sems, send_sems, recv_sems):
        i = lax.axis_index("i")
        p = lax.rem(i, 4)
        bx = jnp.where((p == 1) | (p == 2), 1, 0)
        by = jnp.where(p >= 2, 1, 0)
        bz = i // 4
        part_x = bz * 4 + (p ^ 1)
        part_y = bz * 4 + (3 - p)
        part_z = i ^ 4

        wq_dma = pltpu.make_async_copy(
            wq_ref.at[:, pl.ds(i * D_MODEL, D_MODEL)], wq_v, dma_sems.at[0])
        wq_dma.start()
        wo_dma = pltpu.make_async_copy(
            wo_ref.at[pl.ds(i * D_MODEL, D_MODEL), :], wo_v, dma_sems.at[1])
        wo_dma.start()

        barrier_sem = pltpu.get_barrier_semaphore()
        for prt in (part_x, part_y, part_z):
            pl.semaphore_signal(barrier_sem, inc=1, device_id=(prt,),
                                device_id_type=pl.DeviceIdType.MESH)
        pl.semaphore_wait(barrier_sem, 3)

        wq_dma.wait()
        q_sc[:, :] = jnp.dot(
            x_ref[:, :].astype(jnp.bfloat16),
            wq_v[:, :].astype(jnp.bfloat16),
            preferred_element_type=jnp.float32).astype(jnp.bfloat16)

        for qb in range(N_QB):
            r0 = qb * QB
            kw = min(max(r0 - 256, 0), SQ - KW)
            qi = r0 + lax.broadcasted_iota(jnp.int32, (QB, KW), 0)
            ki = kw + lax.broadcasted_iota(jnp.int32, (QB, KW), 1)
            mask = jnp.abs(qi - ki) <= WINDOW
            for h in range(H_LOC):
                c0 = h * DH
                q_h = q_sc[r0:r0 + QB, c0:c0 + DH]
                k_h = k_ref[kw:kw + KW, c0:c0 + DH].astype(jnp.bfloat16)
                v_h = v_ref[kw:kw + KW, c0:c0 + DH].astype(jnp.bfloat16)
                s = lax.dot_general(
                    q_h, k_h, (((1,), (1,)), ((), ())),
                    preferred_element_type=jnp.float32) * SCALE
                s = jnp.where(mask, s, -1e9)
                m = jnp.max(s, axis=1, keepdims=True)
                w = jnp.exp(s - m)
                w = (w / jnp.sum(w, axis=1, keepdims=True)).astype(jnp.bfloat16)
                ctx_sc[r0:r0 + QB, c0:c0 + DH] = jnp.dot(
                    w, v_h, preferred_element_type=jnp.float32
                    ).astype(jnp.bfloat16)

        wo_dma.wait()
        comm[:, :] = jnp.dot(ctx_sc[:, :], wo_v[:, :].astype(jnp.bfloat16),
                             preferred_element_type=jnp.float32
                             ).astype(jnp.bfloat16)

        half = SQ // 2
        quar = SQ // 4
        eigh = SQ // 8
        rs_plan = [
            (part_x, bx * half, (1 - bx) * half, half, 0),
            (part_y, bx * half + by * quar,
             bx * half + (1 - by) * quar, quar, half),
            (part_z, bx * half + by * quar + bz * eigh,
             bx * half + by * quar + (1 - bz) * eigh, eigh, half + quar),
        ]
        for ph, (prt, keep_off, send_off, size, slot) in enumerate(rs_plan):
            rdma = pltpu.make_async_remote_copy(
                src_ref=comm.at[pl.ds(send_off, size), :],
                dst_ref=rs_recv.at[pl.ds(slot, size), :],
                send_sem=send_sems.at[ph],
                recv_sem=recv_sems.at[ph],
                device_id=(prt,),
                device_id_type=pl.DeviceIdType.MESH,
            )
            rdma.start()
            rdma.wait()
            rows = pl.ds(keep_off, size)
            comm[rows, :] = comm[rows, :] + rs_recv[pl.ds(slot, size), :]

        ag_plan = [
            (part_z, bx * half + by * quar + bz * eigh, eigh),
            (part_y, bx * half + by * quar, quar),
            (part_x, bx * half, half),
        ]
        for ph, (prt, off, size) in enumerate(ag_plan):
            rows = pl.ds(off, size)
            rdma = pltpu.make_async_remote_copy(
                src_ref=comm.at[rows, :],
                dst_ref=comm.at[rows, :],
                send_sem=send_sems.at[3 + ph],
                recv_sem=recv_sems.at[3 + ph],
                device_id=(prt,),
                device_id_type=pl.DeviceIdType.MESH,
            )
            rdma.start()
            rdma.wait()

        out_ref[:, :] = comm[:, :].astype(jnp.float32)

    out = pl.pallas_call(
        body,
        out_shape=jax.ShapeDtypeStruct((SQ, D_MODEL), jnp.float32),
        in_specs=[
            pl.BlockSpec(memory_space=pltpu.VMEM),
            pl.BlockSpec(memory_space=pl.ANY),
            pl.BlockSpec(memory_space=pltpu.VMEM),
            pl.BlockSpec(memory_space=pltpu.VMEM),
            pl.BlockSpec(memory_space=pl.ANY),
        ],
        out_specs=pl.BlockSpec(memory_space=pltpu.VMEM),
        scratch_shapes=[
            pltpu.VMEM((D_MODEL, D_MODEL), jnp.float32),
            pltpu.VMEM((D_MODEL, D_MODEL), jnp.float32),
            pltpu.VMEM((SQ, D_MODEL), jnp.bfloat16),
            pltpu.VMEM((SQ, D_MODEL), jnp.bfloat16),
            pltpu.VMEM((SQ, D_MODEL), jnp.bfloat16),
            pltpu.VMEM((SQ // 2 + SQ // 4 + SQ // 8, D_MODEL),
                       jnp.bfloat16),
            pltpu.SemaphoreType.DMA((2,)),
            pltpu.SemaphoreType.DMA((6,)),
            pltpu.SemaphoreType.DMA((6,)),
        ],
        compiler_params=pltpu.CompilerParams(
            collective_id=0,
            vmem_limit_bytes=120 * 1024 * 1024,
        ),
    )(x2, Wq, K2, V2, Wo)
    return out.reshape(1, SQ, D_MODEL)
